# baseline (speedup 1.0000x reference)
import functools

import jax
import jax.numpy as jnp
from jax import lax
from jax.experimental import pallas as pl
from jax.experimental.pallas import tpu as pltpu
from jax.experimental.pallas import tpu_sc as plsc

B, C, H, W = 4, 96, 384, 384
HW = H * W
N = B * HW
NC, NS = 2, 16
R = 32
BAND = R * W
NPASS = H // (R * NC)
PXT = HW // NS
NB = PXT // 16
ROWS_T = PXT // W
CB = W // 16
SLICE = BAND // NS
ZR = 64
PC = 128


def _warp_sc(src, fx, fy):
  mesh = plsc.VectorSubcoreMesh(core_axis_name="c", subcore_axis_name="s")

  @functools.partial(
      pl.kernel,
      out_type=jax.ShapeDtypeStruct((N, PC), jnp.float32),
      mesh=mesh,
      scratch_types=dict(
          fxv=pltpu.VMEM((PXT,), jnp.float32),
          fyv=pltpu.VMEM((PXT,), jnp.float32),
          rows=pltpu.VMEM((16, C), jnp.float32),
          wbuf=pltpu.VMEM((16, PC), jnp.float32),
          zbuf=pltpu.VMEM((ZR, PC), jnp.float32),
          ymm=pltpu.SMEM((2 * NB,), jnp.float32),
          acc=pltpu.VMEM_SHARED((BAND, PC), jnp.float32),
      ),
  )
  def warp(src_hbm, fx_hbm, fy_hbm, out_hbm, *, fxv, fyv, rows, wbuf,
           zbuf, ymm, acc):
    c = lax.axis_index("c")
    s = lax.axis_index("s")
    iota_i = lax.iota(jnp.int32, 16)
    iota_f = iota_i.astype(jnp.float32)
    zeros16f = jnp.zeros((16,), jnp.float32)

    def _z(i, _):
      for k in range(PC // 16):
        zbuf[i, pl.ds(k * 16, 16)] = zeros16f
      return 0
    lax.fori_loop(0, ZR, _z, 0)
    for l in range(16):
      for k in range(C // 16, PC // 16):
        wbuf[l, pl.ds(k * 16, 16)] = zeros16f

    def scale_scatter(w, dst):
      for l in range(16):
        w_l = w[l]
        for k in range(C // 16):
          csl = pl.ds(k * 16, 16)
          wbuf[l, csl] = rows[l, csl] * w_l
      pltpu.sync_copy(wbuf, acc.at[dst], add=True)

    def batch_body(b, _):
      base_b = b * HW
      pltpu.sync_copy(fx_hbm.at[pl.ds(base_b + s * PXT, PXT)], fxv)
      pltpu.sync_copy(fy_hbm.at[pl.ds(base_b + s * PXT, PXT)], fyv)

      def mm_body(i, _):
        v = fyv[pl.ds(16 * i, 16)]
        lo = v[0]
        hi = v[0]
        for l in range(1, 16):
          v_l = v[l]
          lo = jnp.minimum(lo, v_l)
          hi = jnp.maximum(hi, v_l)
        ymm[2 * i] = lo
        ymm[2 * i + 1] = hi
        return 0
      lax.fori_loop(0, NB, mm_body, 0)

      def pass_body(p, _):
        row0 = (p * NC + c) * R
        row0_f = row0.astype(jnp.float32)
        for j in range(SLICE // ZR):
          pltpu.sync_copy(zbuf, acc.at[pl.ds(s * SLICE + j * ZR, ZR)])
        plsc.subcore_barrier()

        def blk_body(i, _):
          rr = i // CB
          cb = i - rr * CB
          gy_f = (s * ROWS_T + rr).astype(jnp.float32)
          ymin = gy_f + ymm[2 * i]
          ymax = gy_f + ymm[2 * i + 1]
          hit = (ymax >= row0_f - 1.0) & (ymin < row0_f + R)

          @pl.when(hit)
          def _do():
            off = i * 16
            fxb = fxv[pl.ds(off, 16)]
            fyb = fyv[pl.ds(off, 16)]
            x = (cb * 16).astype(jnp.float32) + iota_f + fxb
            y = gy_f + fyb
            x = jnp.clip(x, -2.0, 385.0)
            y = jnp.clip(y, -2.0, 385.0)
            xt = x.astype(jnp.int32)
            xf_i = jnp.where(xt.astype(jnp.float32) > x, xt - 1, xt)
            yt = y.astype(jnp.int32)
            yf_i = jnp.where(yt.astype(jnp.float32) > y, yt - 1, yt)
            dx = x - xf_i.astype(jnp.float32)
            dy = y - yf_i.astype(jnp.float32)
            omdx = 1.0 - dx
            omdy = 1.0 - dy
            valid = ((xf_i >= 0) & (xf_i <= W - 2)
                     & (yf_i >= 0) & (yf_i <= H - 2))
            yrel = yf_i - row0
            mask_n = valid & (yrel >= 0) & (yrel < R)
            mask_s = valid & (yrel >= -1) & (yrel < R - 1)

            pltpu.sync_copy(src_hbm.at[pl.ds(base_b + s * PXT + off, 16)],
                            rows)
            zi = jnp.zeros((16,), jnp.int32)
            dn = jnp.where(mask_n, yrel * W + xf_i, zi)
            ds_ = jnp.where(mask_s, (yrel + 1) * W + xf_i, zi)
            scale_scatter(jnp.where(mask_n, omdx * omdy, zeros16f), dn)
            scale_scatter(jnp.where(mask_n, dx * omdy, zeros16f), dn + 1)
            scale_scatter(jnp.where(mask_s, omdx * dy, zeros16f), ds_)
            scale_scatter(jnp.where(mask_s, dx * dy, zeros16f), ds_ + 1)

          return 0

        lax.fori_loop(0, NB, blk_body, 0)
        plsc.subcore_barrier()
        out_base = base_b + row0 * W + s * SLICE
        pltpu.sync_copy(acc.at[pl.ds(s * SLICE, SLICE)],
                        out_hbm.at[pl.ds(out_base, SLICE)])
        return 0

      lax.fori_loop(0, NPASS, pass_body, 0)
      return 0

    lax.fori_loop(0, B, batch_body, 0)

  return warp(src, fx, fy)


def kernel(im0, flow):
  src = jnp.transpose(im0, (0, 2, 3, 1)).reshape(N, C)
  fx = flow[..., 0].reshape(N)
  fy = flow[..., 1].reshape(N)
  out = _warp_sc(src, fx, fy)
  return out.reshape(B, H, W, PC)[..., :C].transpose(0, 3, 1, 2)


# speedup vs baseline: 1.1463x; 1.1463x over previous
import functools

import jax
import jax.numpy as jnp
from jax import lax
from jax.experimental import pallas as pl
from jax.experimental.pallas import tpu as pltpu
from jax.experimental.pallas import tpu_sc as plsc

B, C, H, W = 4, 96, 384, 384
HW = H * W
N = B * HW
NC, NS = 2, 16
R = 32
BAND = R * W
NPASS = H // (R * NC)
PXT = HW // NS
NB = PXT // 16
ROWS_T = PXT // W
CB = W // 16
SLICE = BAND // NS
ZR = 16
PC = 128


def _warp_sc(src, fx, fy):
  mesh = plsc.VectorSubcoreMesh(core_axis_name="c", subcore_axis_name="s")

  @functools.partial(
      pl.kernel,
      out_type=jax.ShapeDtypeStruct((N, PC), jnp.float32),
      mesh=mesh,
      scratch_types=dict(
          fxv=pltpu.VMEM((PXT,), jnp.float32),
          fyv=pltpu.VMEM((PXT,), jnp.float32),
          rows=pltpu.VMEM((16, C), jnp.float32),
          wbuf0=pltpu.VMEM((16, PC), jnp.float32),
          wbuf1=pltpu.VMEM((16, PC), jnp.float32),
          wbuf2=pltpu.VMEM((16, PC), jnp.float32),
          wbuf3=pltpu.VMEM((16, PC), jnp.float32),
          zbuf=pltpu.VMEM((ZR, PC), jnp.float32),
          ymm=pltpu.SMEM((2 * NB,), jnp.float32),
          acc=pltpu.VMEM_SHARED((BAND, PC), jnp.float32),
          gsem=pltpu.SemaphoreType.DMA,
          ssem=pltpu.SemaphoreType.DMA,
      ),
  )
  def warp(src_hbm, fx_hbm, fy_hbm, out_hbm, *, fxv, fyv, rows, wbuf0,
           wbuf1, wbuf2, wbuf3, zbuf, ymm, acc, gsem, ssem):
    c = lax.axis_index("c")
    s = lax.axis_index("s")
    iota_i = lax.iota(jnp.int32, 16)
    iota_f = iota_i.astype(jnp.float32)
    zeros16f = jnp.zeros((16,), jnp.float32)

    def _z(i, _):
      for k in range(PC // 16):
        zbuf[i, pl.ds(k * 16, 16)] = zeros16f
      return 0
    lax.fori_loop(0, ZR, _z, 0)
    for wb in (wbuf0, wbuf1, wbuf2, wbuf3):
      for l in range(16):
        for k in range(C // 16, PC // 16):
          wb[l, pl.ds(k * 16, 16)] = zeros16f

    def scale(wb, w):
      for l in range(16):
        w_l = w[l]
        for k in range(C // 16):
          csl = pl.ds(k * 16, 16)
          wb[l, csl] = rows[l, csl] * w_l

    def batch_body(b, _):
      base_b = b * HW
      pltpu.sync_copy(fx_hbm.at[pl.ds(base_b + s * PXT, PXT)], fxv)
      pltpu.sync_copy(fy_hbm.at[pl.ds(base_b + s * PXT, PXT)], fyv)

      def mm_body(i, _):
        v = fyv[pl.ds(16 * i, 16)]
        lo = v[0]
        hi = v[0]
        for l in range(1, 16):
          v_l = v[l]
          lo = jnp.minimum(lo, v_l)
          hi = jnp.maximum(hi, v_l)
        ymm[2 * i] = lo
        ymm[2 * i + 1] = hi
        return 0
      lax.fori_loop(0, NB, mm_body, 0)

      def pass_body(p, _):
        row0 = (p * NC + c) * R
        row0_f = row0.astype(jnp.float32)
        for j in range(SLICE // ZR):
          pltpu.sync_copy(zbuf, acc.at[pl.ds(s * SLICE + j * ZR, ZR)])
        plsc.subcore_barrier()

        def blk_body(i, _):
          rr = i // CB
          cb = i - rr * CB
          gy_f = (s * ROWS_T + rr).astype(jnp.float32)
          ymin = gy_f + ymm[2 * i]
          ymax = gy_f + ymm[2 * i + 1]
          hit = (ymax >= row0_f - 1.0) & (ymin < row0_f + R)

          @pl.when(hit)
          def _do():
            off = i * 16
            grab = pltpu.async_copy(
                src_hbm.at[pl.ds(base_b + s * PXT + off, 16)], rows, gsem)
            fxb = fxv[pl.ds(off, 16)]
            fyb = fyv[pl.ds(off, 16)]
            x = (cb * 16).astype(jnp.float32) + iota_f + fxb
            y = gy_f + fyb
            x = jnp.clip(x, -2.0, 385.0)
            y = jnp.clip(y, -2.0, 385.0)
            xt = x.astype(jnp.int32)
            xf_i = jnp.where(xt.astype(jnp.float32) > x, xt - 1, xt)
            yt = y.astype(jnp.int32)
            yf_i = jnp.where(yt.astype(jnp.float32) > y, yt - 1, yt)
            dx = x - xf_i.astype(jnp.float32)
            dy = y - yf_i.astype(jnp.float32)
            omdx = 1.0 - dx
            omdy = 1.0 - dy
            valid = ((xf_i >= 0) & (xf_i <= W - 2)
                     & (yf_i >= 0) & (yf_i <= H - 2))
            yrel = yf_i - row0
            mask_n = valid & (yrel >= 0) & (yrel < R)
            mask_s = valid & (yrel >= -1) & (yrel < R - 1)

            zi = jnp.zeros((16,), jnp.int32)
            dn = jnp.where(mask_n, yrel * W + xf_i, zi)
            ds_ = jnp.where(mask_s, (yrel + 1) * W + xf_i, zi)
            grab.wait()
            scale(wbuf0, jnp.where(mask_n, omdx * omdy, zeros16f))
            scale(wbuf1, jnp.where(mask_n, dx * omdy, zeros16f))
            scale(wbuf2, jnp.where(mask_s, omdx * dy, zeros16f))
            scale(wbuf3, jnp.where(mask_s, dx * dy, zeros16f))
            h0 = pltpu.async_copy(wbuf0, acc.at[dn], ssem, add=True)
            h1 = pltpu.async_copy(wbuf1, acc.at[dn + 1], ssem, add=True)
            h2 = pltpu.async_copy(wbuf2, acc.at[ds_], ssem, add=True)
            h3 = pltpu.async_copy(wbuf3, acc.at[ds_ + 1], ssem, add=True)
            h0.wait()
            h1.wait()
            h2.wait()
            h3.wait()

          return 0

        lax.fori_loop(0, NB, blk_body, 0)
        plsc.subcore_barrier()
        out_base = base_b + row0 * W + s * SLICE
        pltpu.sync_copy(acc.at[pl.ds(s * SLICE, SLICE)],
                        out_hbm.at[pl.ds(out_base, SLICE)])
        return 0

      lax.fori_loop(0, NPASS, pass_body, 0)
      return 0

    lax.fori_loop(0, B, batch_body, 0)

  return warp(src, fx, fy)


def kernel(im0, flow):
  src = jnp.transpose(im0, (0, 2, 3, 1)).reshape(N, C)
  fx = flow[..., 0].reshape(N)
  fy = flow[..., 1].reshape(N)
  out = _warp_sc(src, fx, fy)
  return out.reshape(B, H, W, PC)[..., :C].transpose(0, 3, 1, 2)
